# baseline (speedup 1.0000x reference)
import functools

import jax
import jax.numpy as jnp
from jax import lax
from jax.experimental import pallas as pl
from jax.experimental.pallas import tpu as pltpu
from jax.experimental.pallas import tpu_sc as plsc

N = 10000
E = 320000
D_IN = 128
D_HID = 128
D_OUT = 40
D_OUT_PAD = 48

NC = 2
NS = 16
NW = NC * NS
NPAD = 10240
STRIPE = NPAD // NS
EPW = E // NW
K = 80
NCHUNK = EPW // K
NPAIR = NCHUNK // 2

_mesh = plsc.VectorSubcoreMesh(core_axis_name="c", subcore_axis_name="s")


@functools.partial(
    pl.kernel,
    out_type=jax.ShapeDtypeStruct((NC, NPAD), jnp.float32),
    mesh=_mesh,
    scratch_types=[
        pltpu.VMEM((NCHUNK, K), jnp.int32),
        pltpu.VMEM((NCHUNK, K), jnp.float32),
        pltpu.VMEM((STRIPE,), jnp.float32),
        pltpu.VMEM_SHARED((NPAD,), jnp.float32),
        pltpu.SemaphoreType.DMA,
    ],
    compiler_params=pltpu.CompilerParams(use_tc_tiling_on_sc=False),
)
def _deg_kernel(col_hbm, ew_hbm, out_hbm, colm, ewm, zb, deg_sh, sd):
    c = lax.axis_index("c")
    s = lax.axis_index("s")
    wid = s * NC + c
    rbase = s * STRIPE
    cbase = wid * NCHUNK
    pltpu.sync_copy(col_hbm.at[pl.ds(cbase, NCHUNK)], colm)
    pltpu.sync_copy(ew_hbm.at[pl.ds(cbase, NCHUNK)], ewm)
    for i in range(STRIPE // 16):
        zb[pl.ds(i * 16, 16)] = jnp.zeros((16,), jnp.float32)
    pltpu.sync_copy(zb, deg_sh.at[pl.ds(rbase, STRIPE)])
    plsc.subcore_barrier()

    FB = 8

    def group(jj, carry):
        j0 = jj * FB
        for i in range(FB):
            pltpu.async_copy(ewm.at[j0 + i], deg_sh.at[colm.at[j0 + i]], sd,
                             add=True)
        for i in range(FB):
            pltpu.make_async_copy(ewm.at[j0 + i],
                                  deg_sh.at[colm.at[j0 + i]], sd).wait()
        return carry

    lax.fori_loop(0, NCHUNK // FB, group, 0)
    for j in range(NCHUNK - NCHUNK % FB, NCHUNK):
        pltpu.async_copy(ewm.at[j], deg_sh.at[colm.at[j]], sd, add=True)
    for j in range(NCHUNK - NCHUNK % FB, NCHUNK):
        pltpu.make_async_copy(ewm.at[j], deg_sh.at[colm.at[j]], sd).wait()
    plsc.subcore_barrier()
    pltpu.sync_copy(
        deg_sh.at[pl.ds(rbase, STRIPE)], out_hbm.at[c, pl.ds(rbase, STRIPE)]
    )


def _make_agg(D, pages):
    G = D // 16
    PGMAX = max(p for _, p in pages)

    @functools.partial(
        pl.kernel,
        out_type=jax.ShapeDtypeStruct((NC, NPAD, D), jnp.float32),
        mesh=_mesh,
        scratch_types=[
            pltpu.VMEM((PGMAX, K), jnp.int32),
            pltpu.VMEM((PGMAX, K), jnp.int32),
            pltpu.VMEM((PGMAX, K), jnp.float32),
            pltpu.VMEM((K, D), jnp.float32),
            pltpu.VMEM((K, D), jnp.float32),
            pltpu.VMEM((K, D), jnp.float32),
            pltpu.VMEM_SHARED((NPAD, D), jnp.float32),
            pltpu.SemaphoreType.DMA,
            pltpu.SemaphoreType.DMA,
            pltpu.SemaphoreType.DMA,
            pltpu.SemaphoreType.DMA,
            pltpu.SemaphoreType.DMA,
            pltpu.SemaphoreType.DMA,
        ],
        compiler_params=pltpu.CompilerParams(use_tc_tiling_on_sc=False),
    )
    def _agg(h_hbm, row_hbm, col_hbm, ew_hbm, out_hbm, rowm, colm, ewm,
             b0, b1, b2, acc_sh, sg0, sg1, sg2, ss0, ss1, ss2):
        c = lax.axis_index("c")
        s = lax.axis_index("s")
        wid = s * NC + c
        rbase = s * STRIPE
        cbase = wid * NCHUNK
        pltpu.sync_copy(
            h_hbm.at[pl.ds(rbase, STRIPE)], acc_sh.at[pl.ds(rbase, STRIPE)]
        )
        plsc.subcore_barrier()

        def g_start(j, buf, sem):
            pltpu.async_copy(h_hbm.at[rowm.at[j]], buf, sem)

        def g_wait(j, buf, sem):
            pltpu.make_async_copy(h_hbm.at[rowm.at[j]], buf, sem).wait()

        def s_start(j, buf, sem):
            pltpu.async_copy(buf, acc_sh.at[colm.at[j]], sem, add=True)

        def s_wait(j, buf, sem):
            pltpu.make_async_copy(buf, acc_sh.at[colm.at[j]], sem).wait()

        def scale(j, buf):
            @plsc.parallel_loop(0, K // 16, unroll=K // 16)
            def srow(t):
                ew16 = ewm[j, pl.ds(t * 16, 16)]
                for kk in range(16):
                    w = ew16[kk]
                    k = t * 16 + kk
                    for g in range(G):
                        buf[k, pl.ds(g * 16, 16)] = (
                            buf[k, pl.ds(g * 16, 16)] * w
                        )

        B = (b0, b1, b2)
        SG = (sg0, sg1, sg2)
        SS = (ss0, ss1, ss2)

        for c0, P in pages:
            pltpu.sync_copy(row_hbm.at[pl.ds(cbase + c0, P)],
                            rowm.at[pl.ds(0, P)])
            pltpu.sync_copy(col_hbm.at[pl.ds(cbase + c0, P)],
                            colm.at[pl.ds(0, P)])
            pltpu.sync_copy(ew_hbm.at[pl.ds(cbase + c0, P)],
                            ewm.at[pl.ds(0, P)])
            g_start(0, B[0], SG[0])

            def step(j, u):
                g_wait(j, B[u], SG[u])

                @pl.when(j >= 2)
                def _():
                    s_wait(j - 2, B[(u + 1) % 3], SS[(u + 1) % 3])

                @pl.when(j + 1 < P)
                def _():
                    g_start(j + 1, B[(u + 1) % 3], SG[(u + 1) % 3])

                scale(j, B[u])
                s_start(j, B[u], SS[u])

            def triple(jj, carry):
                for u in range(3):
                    step(3 * jj + u, u)
                return carry

            NT = P // 3
            lax.fori_loop(0, NT, triple, 0)
            for u in range(P - 3 * NT):
                step(3 * NT + u, u)
            s_wait(P - 2, B[(P - 2) % 3], SS[(P - 2) % 3])
            s_wait(P - 1, B[(P - 1) % 3], SS[(P - 1) % 3])

        plsc.subcore_barrier()
        pltpu.sync_copy(
            acc_sh.at[pl.ds(rbase, STRIPE)], out_hbm.at[c, pl.ds(rbase, STRIPE)]
        )

    return _agg


_agg128 = _make_agg(D_HID, pages=[(0, 63), (63, 62)])
_agg48 = _make_agg(D_OUT_PAD, pages=[(0, NCHUNK)])


BLK = 512
GRID = NPAD // BLK


def _lin1_body(x_ref, w_ref, d0_ref, d1_ref, h_ref, dis_ref):
    deg = d0_ref[...] + d1_ref[...] + 1.0
    dis = jnp.where(deg > 0, lax.rsqrt(deg), 0.0)
    h = lax.dot_general(
        x_ref[...], w_ref[...],
        (((1,), (1,)), ((), ())),
        preferred_element_type=jnp.float32,
    )
    h_ref[...] = h * dis
    dis_ref[...] = dis


def _lin1_call(xp, W1, d0, d1):
    return pl.pallas_call(
        _lin1_body,
        grid=(GRID,),
        in_specs=[
            pl.BlockSpec((BLK, D_IN), lambda i: (i, 0)),
            pl.BlockSpec((D_HID, D_IN), lambda i: (0, 0)),
            pl.BlockSpec((BLK, 1), lambda i: (i, 0)),
            pl.BlockSpec((BLK, 1), lambda i: (i, 0)),
        ],
        out_specs=[
            pl.BlockSpec((BLK, D_HID), lambda i: (i, 0)),
            pl.BlockSpec((BLK, 1), lambda i: (i, 0)),
        ],
        out_shape=[
            jax.ShapeDtypeStruct((NPAD, D_HID), jnp.float32),
            jax.ShapeDtypeStruct((NPAD, 1), jnp.float32),
        ],
    )(xp, W1, d0, d1)


def _mid_body(p0_ref, p1_ref, h1_ref, dis_ref, w2_ref, b1_ref, h2_ref):
    dis = dis_ref[...]
    z = (p0_ref[...] + p1_ref[...] - h1_ref[...]) * dis + b1_ref[...]
    z = jnp.maximum(z, 0.0)
    h2 = lax.dot_general(
        z, w2_ref[...],
        (((1,), (1,)), ((), ())),
        preferred_element_type=jnp.float32,
    )
    h2_ref[...] = h2 * dis


def _mid_call(p0, p1, h1, dis, W2p, b1r):
    return pl.pallas_call(
        _mid_body,
        grid=(GRID,),
        in_specs=[
            pl.BlockSpec((BLK, D_HID), lambda i: (i, 0)),
            pl.BlockSpec((BLK, D_HID), lambda i: (i, 0)),
            pl.BlockSpec((BLK, D_HID), lambda i: (i, 0)),
            pl.BlockSpec((BLK, 1), lambda i: (i, 0)),
            pl.BlockSpec((D_OUT_PAD, D_HID), lambda i: (0, 0)),
            pl.BlockSpec((1, D_HID), lambda i: (0, 0)),
        ],
        out_specs=pl.BlockSpec((BLK, D_OUT_PAD), lambda i: (i, 0)),
        out_shape=jax.ShapeDtypeStruct((NPAD, D_OUT_PAD), jnp.float32),
    )(p0, p1, h1, dis, W2p, b1r)


def _out_body(p0_ref, p1_ref, h2_ref, dis_ref, b2_ref, o_ref):
    o_ref[...] = (
        (p0_ref[...] + p1_ref[...] - h2_ref[...]) * dis_ref[...] + b2_ref[...]
    )


def _out_call(p0, p1, h2, dis, b2r):
    return pl.pallas_call(
        _out_body,
        grid=(GRID,),
        in_specs=[
            pl.BlockSpec((BLK, D_OUT_PAD), lambda i: (i, 0)),
            pl.BlockSpec((BLK, D_OUT_PAD), lambda i: (i, 0)),
            pl.BlockSpec((BLK, D_OUT_PAD), lambda i: (i, 0)),
            pl.BlockSpec((BLK, 1), lambda i: (i, 0)),
            pl.BlockSpec((1, D_OUT_PAD), lambda i: (0, 0)),
        ],
        out_specs=pl.BlockSpec((BLK, D_OUT_PAD), lambda i: (i, 0)),
        out_shape=jax.ShapeDtypeStruct((NPAD, D_OUT_PAD), jnp.float32),
    )(p0, p1, h2, dis, b2r)


def kernel(x, edge_index, edge_weight, W1, b1, W2, b2):
    xp = jnp.pad(x, ((0, NPAD - N), (0, 0)))
    row2 = edge_index[0].reshape(E // K, K)
    col2 = edge_index[1].reshape(E // K, K)
    ew2 = edge_weight.reshape(E // K, K)

    degp = _deg_kernel(col2, ew2)
    d0 = degp[0].reshape(NPAD, 1)
    d1 = degp[1].reshape(NPAD, 1)

    h1, dis = _lin1_call(xp, W1, d0, d1)
    p1 = _agg128(h1, row2, col2, ew2)

    W2p = jnp.pad(W2, ((0, D_OUT_PAD - D_OUT), (0, 0)))
    h2 = _mid_call(p1[0], p1[1], h1, dis, W2p, b1.reshape(1, -1))

    p2 = _agg48(h2, row2, col2, ew2)
    b2p = jnp.pad(b2, (0, D_OUT_PAD - D_OUT)).reshape(1, -1)
    out = _out_call(p2[0], p2[1], h2, dis, b2p)
    return out[:N, :D_OUT]


# speedup vs baseline: 1.2106x; 1.2106x over previous
import functools

import jax
import jax.numpy as jnp
from jax import lax
from jax.experimental import pallas as pl
from jax.experimental.pallas import tpu as pltpu
from jax.experimental.pallas import tpu_sc as plsc

N = 10000
E = 320000
D_IN = 128
D_HID = 128
D_OUT = 40
D_OUT_PAD = 48

NC = 2
NS = 16
NW = NC * NS
NPAD = 10240
STRIPE = NPAD // NS
EPW = E // NW
K = 80
NCHUNK = EPW // K
NPAIR = NCHUNK // 2

_mesh = plsc.VectorSubcoreMesh(core_axis_name="c", subcore_axis_name="s")


@functools.partial(
    pl.kernel,
    out_type=jax.ShapeDtypeStruct((NC, NPAD), jnp.float32),
    mesh=_mesh,
    scratch_types=[
        pltpu.VMEM((NCHUNK, K), jnp.int32),
        pltpu.VMEM((NCHUNK, K), jnp.float32),
        pltpu.VMEM((STRIPE,), jnp.float32),
        pltpu.VMEM_SHARED((NPAD,), jnp.float32),
        pltpu.SemaphoreType.DMA,
    ],
    compiler_params=pltpu.CompilerParams(use_tc_tiling_on_sc=False),
)
def _deg_kernel(col_hbm, ew_hbm, out_hbm, colm, ewm, zb, deg_sh, sd):
    c = lax.axis_index("c")
    s = lax.axis_index("s")
    wid = s * NC + c
    rbase = s * STRIPE
    cbase = wid * NCHUNK
    pltpu.sync_copy(col_hbm.at[pl.ds(cbase, NCHUNK)], colm)
    pltpu.sync_copy(ew_hbm.at[pl.ds(cbase, NCHUNK)], ewm)
    for i in range(STRIPE // 16):
        zb[pl.ds(i * 16, 16)] = jnp.zeros((16,), jnp.float32)
    pltpu.sync_copy(zb, deg_sh.at[pl.ds(rbase, STRIPE)])
    plsc.subcore_barrier()

    FB = 8

    def group(jj, carry):
        j0 = jj * FB
        for i in range(FB):
            pltpu.async_copy(ewm.at[j0 + i], deg_sh.at[colm.at[j0 + i]], sd,
                             add=True)
        for i in range(FB):
            pltpu.make_async_copy(ewm.at[j0 + i],
                                  deg_sh.at[colm.at[j0 + i]], sd).wait()
        return carry

    lax.fori_loop(0, NCHUNK // FB, group, 0)
    for j in range(NCHUNK - NCHUNK % FB, NCHUNK):
        pltpu.async_copy(ewm.at[j], deg_sh.at[colm.at[j]], sd, add=True)
    for j in range(NCHUNK - NCHUNK % FB, NCHUNK):
        pltpu.make_async_copy(ewm.at[j], deg_sh.at[colm.at[j]], sd).wait()
    plsc.subcore_barrier()
    pltpu.sync_copy(
        deg_sh.at[pl.ds(rbase, STRIPE)], out_hbm.at[c, pl.ds(rbase, STRIPE)]
    )


def _make_agg(D, pages):
    G = D // 16
    PGMAX = max(p for _, p in pages)

    @functools.partial(
        pl.kernel,
        out_type=jax.ShapeDtypeStruct((NC, NPAD, D), jnp.float32),
        mesh=_mesh,
        scratch_types=[
            pltpu.VMEM((PGMAX, K), jnp.int32),
            pltpu.VMEM((PGMAX, K), jnp.int32),
            pltpu.VMEM((PGMAX, K), jnp.float32),
            pltpu.VMEM((K, D), jnp.float32),
            pltpu.VMEM((K, D), jnp.float32),
            pltpu.VMEM((K, D), jnp.float32),
            pltpu.VMEM((K, D), jnp.float32),
            pltpu.VMEM_SHARED((NPAD, D), jnp.float32),
            pltpu.SemaphoreType.DMA,
            pltpu.SemaphoreType.DMA,
            pltpu.SemaphoreType.DMA,
            pltpu.SemaphoreType.DMA,
        ],
        compiler_params=pltpu.CompilerParams(use_tc_tiling_on_sc=False),
    )
    def _agg(h_hbm, row_hbm, col_hbm, ew_hbm, out_hbm, rowm, colm, ewm,
             gb0, gb1, sb0, sb1, acc_sh, sg0, sg1, ss0, ss1):
        c = lax.axis_index("c")
        s = lax.axis_index("s")
        wid = s * NC + c
        rbase = s * STRIPE
        cbase = wid * NCHUNK
        pltpu.sync_copy(
            h_hbm.at[pl.ds(rbase, STRIPE)], acc_sh.at[pl.ds(rbase, STRIPE)]
        )
        plsc.subcore_barrier()

        def g_start(j, buf, sem):
            pltpu.async_copy(h_hbm.at[rowm.at[j]], buf, sem)

        def g_wait(j, buf, sem):
            pltpu.make_async_copy(h_hbm.at[rowm.at[j]], buf, sem).wait()

        def s_start(j, buf, sem):
            pltpu.async_copy(buf, acc_sh.at[colm.at[j]], sem, add=True)

        def s_wait(j, buf, sem):
            pltpu.make_async_copy(buf, acc_sh.at[colm.at[j]], sem).wait()

        def scale(j, src, dst):
            @plsc.parallel_loop(0, K // 16, unroll=K // 16)
            def srow(t):
                ew16 = ewm[j, pl.ds(t * 16, 16)]
                for kk in range(16):
                    w = ew16[kk]
                    k = t * 16 + kk
                    for g in range(G):
                        dst[k, pl.ds(g * 16, 16)] = (
                            src[k, pl.ds(g * 16, 16)] * w
                        )

        GB = (gb0, gb1)
        SB = (sb0, sb1)
        SG = (sg0, sg1)
        SS = (ss0, ss1)

        for c0, P in pages:
            pltpu.sync_copy(row_hbm.at[pl.ds(cbase + c0, P)],
                            rowm.at[pl.ds(0, P)])
            pltpu.sync_copy(col_hbm.at[pl.ds(cbase + c0, P)],
                            colm.at[pl.ds(0, P)])
            pltpu.sync_copy(ew_hbm.at[pl.ds(cbase + c0, P)],
                            ewm.at[pl.ds(0, P)])
            g_start(0, GB[0], SG[0])
            g_start(1, GB[1], SG[1])

            def step(j, p):
                g_wait(j, GB[p], SG[p])

                @pl.when(j >= 2)
                def _():
                    s_wait(j - 2, SB[p], SS[p])

                scale(j, GB[p], SB[p])

                @pl.when(j + 2 < P)
                def _():
                    g_start(j + 2, GB[p], SG[p])

                s_start(j, SB[p], SS[p])

            def pair(jj, carry):
                step(2 * jj, 0)
                step(2 * jj + 1, 1)
                return carry

            lax.fori_loop(0, P // 2, pair, 0)
            if P % 2:
                step(P - 1, (P - 1) % 2)
            s_wait(P - 2, SB[(P - 2) % 2], SS[(P - 2) % 2])
            s_wait(P - 1, SB[(P - 1) % 2], SS[(P - 1) % 2])

        plsc.subcore_barrier()
        pltpu.sync_copy(
            acc_sh.at[pl.ds(rbase, STRIPE)], out_hbm.at[c, pl.ds(rbase, STRIPE)]
        )

    return _agg


_agg128 = _make_agg(D_HID, pages=[(0, 32), (32, 32), (64, 32), (96, 29)])
_agg48 = _make_agg(D_OUT_PAD, pages=[(0, NCHUNK)])


BLK = 512
GRID = NPAD // BLK


def _lin1_body(x_ref, w_ref, d0_ref, d1_ref, h_ref, dis_ref):
    deg = d0_ref[...] + d1_ref[...] + 1.0
    dis = jnp.where(deg > 0, lax.rsqrt(deg), 0.0)
    h = lax.dot_general(
        x_ref[...], w_ref[...],
        (((1,), (1,)), ((), ())),
        preferred_element_type=jnp.float32,
    )
    h_ref[...] = h * dis
    dis_ref[...] = dis


def _lin1_call(xp, W1, d0, d1):
    return pl.pallas_call(
        _lin1_body,
        grid=(GRID,),
        in_specs=[
            pl.BlockSpec((BLK, D_IN), lambda i: (i, 0)),
            pl.BlockSpec((D_HID, D_IN), lambda i: (0, 0)),
            pl.BlockSpec((BLK, 1), lambda i: (i, 0)),
            pl.BlockSpec((BLK, 1), lambda i: (i, 0)),
        ],
        out_specs=[
            pl.BlockSpec((BLK, D_HID), lambda i: (i, 0)),
            pl.BlockSpec((BLK, 1), lambda i: (i, 0)),
        ],
        out_shape=[
            jax.ShapeDtypeStruct((NPAD, D_HID), jnp.float32),
            jax.ShapeDtypeStruct((NPAD, 1), jnp.float32),
        ],
    )(xp, W1, d0, d1)


def _mid_body(p0_ref, p1_ref, h1_ref, dis_ref, w2_ref, b1_ref, h2_ref):
    dis = dis_ref[...]
    z = (p0_ref[...] + p1_ref[...] - h1_ref[...]) * dis + b1_ref[...]
    z = jnp.maximum(z, 0.0)
    h2 = lax.dot_general(
        z, w2_ref[...],
        (((1,), (1,)), ((), ())),
        preferred_element_type=jnp.float32,
    )
    h2_ref[...] = h2 * dis


def _mid_call(p0, p1, h1, dis, W2p, b1r):
    return pl.pallas_call(
        _mid_body,
        grid=(GRID,),
        in_specs=[
            pl.BlockSpec((BLK, D_HID), lambda i: (i, 0)),
            pl.BlockSpec((BLK, D_HID), lambda i: (i, 0)),
            pl.BlockSpec((BLK, D_HID), lambda i: (i, 0)),
            pl.BlockSpec((BLK, 1), lambda i: (i, 0)),
            pl.BlockSpec((D_OUT_PAD, D_HID), lambda i: (0, 0)),
            pl.BlockSpec((1, D_HID), lambda i: (0, 0)),
        ],
        out_specs=pl.BlockSpec((BLK, D_OUT_PAD), lambda i: (i, 0)),
        out_shape=jax.ShapeDtypeStruct((NPAD, D_OUT_PAD), jnp.float32),
    )(p0, p1, h1, dis, W2p, b1r)


def _out_body(p0_ref, p1_ref, h2_ref, dis_ref, b2_ref, o_ref):
    o_ref[...] = (
        (p0_ref[...] + p1_ref[...] - h2_ref[...]) * dis_ref[...] + b2_ref[...]
    )


def _out_call(p0, p1, h2, dis, b2r):
    return pl.pallas_call(
        _out_body,
        grid=(GRID,),
        in_specs=[
            pl.BlockSpec((BLK, D_OUT_PAD), lambda i: (i, 0)),
            pl.BlockSpec((BLK, D_OUT_PAD), lambda i: (i, 0)),
            pl.BlockSpec((BLK, D_OUT_PAD), lambda i: (i, 0)),
            pl.BlockSpec((BLK, 1), lambda i: (i, 0)),
            pl.BlockSpec((1, D_OUT_PAD), lambda i: (0, 0)),
        ],
        out_specs=pl.BlockSpec((BLK, D_OUT_PAD), lambda i: (i, 0)),
        out_shape=jax.ShapeDtypeStruct((NPAD, D_OUT_PAD), jnp.float32),
    )(p0, p1, h2, dis, b2r)


def kernel(x, edge_index, edge_weight, W1, b1, W2, b2):
    xp = jnp.pad(x, ((0, NPAD - N), (0, 0)))
    row2 = edge_index[0].reshape(E // K, K)
    col2 = edge_index[1].reshape(E // K, K)
    ew2 = edge_weight.reshape(E // K, K)

    degp = _deg_kernel(col2, ew2)
    d0 = degp[0].reshape(NPAD, 1)
    d1 = degp[1].reshape(NPAD, 1)

    h1, dis = _lin1_call(xp, W1, d0, d1)
    p1 = _agg128(h1, row2, col2, ew2)

    W2p = jnp.pad(W2, ((0, D_OUT_PAD - D_OUT), (0, 0)))
    h2 = _mid_call(p1[0], p1[1], h1, dis, W2p, b1.reshape(1, -1))

    p2 = _agg48(h2, row2, col2, ew2)
    b2p = jnp.pad(b2, (0, D_OUT_PAD - D_OUT)).reshape(1, -1)
    out = _out_call(p2[0], p2[1], h2, dis, b2p)
    return out[:N, :D_OUT]


# speedup vs baseline: 1.2273x; 1.0138x over previous
import functools

import jax
import jax.numpy as jnp
from jax import lax
from jax.experimental import pallas as pl
from jax.experimental.pallas import tpu as pltpu
from jax.experimental.pallas import tpu_sc as plsc

N = 10000
E = 320000
D_IN = 128
D_HID = 128
D_OUT = 40
D_OUT_PAD = 48

NC = 2
NS = 16
NW = NC * NS
NPAD = 10240
STRIPE = NPAD // NS
EPW = E // NW
K = 80
NCHUNK = EPW // K
NPAIR = NCHUNK // 2

_mesh = plsc.VectorSubcoreMesh(core_axis_name="c", subcore_axis_name="s")


@functools.partial(
    pl.kernel,
    out_type=jax.ShapeDtypeStruct((NC, NPAD), jnp.float32),
    mesh=_mesh,
    scratch_types=[
        pltpu.VMEM((NCHUNK, K), jnp.int32),
        pltpu.VMEM((NCHUNK, K), jnp.float32),
        pltpu.VMEM((STRIPE,), jnp.float32),
        pltpu.VMEM_SHARED((NPAD,), jnp.float32),
        pltpu.SemaphoreType.DMA,
    ],
    compiler_params=pltpu.CompilerParams(use_tc_tiling_on_sc=False),
)
def _deg_kernel(col_hbm, ew_hbm, out_hbm, colm, ewm, zb, deg_sh, sd):
    c = lax.axis_index("c")
    s = lax.axis_index("s")
    wid = s * NC + c
    rbase = s * STRIPE
    cbase = wid * NCHUNK
    pltpu.sync_copy(col_hbm.at[pl.ds(cbase, NCHUNK)], colm)
    pltpu.sync_copy(ew_hbm.at[pl.ds(cbase, NCHUNK)], ewm)
    for i in range(STRIPE // 16):
        zb[pl.ds(i * 16, 16)] = jnp.zeros((16,), jnp.float32)
    pltpu.sync_copy(zb, deg_sh.at[pl.ds(rbase, STRIPE)])
    plsc.subcore_barrier()

    FB = 8

    def group(jj, carry):
        j0 = jj * FB
        for i in range(FB):
            pltpu.async_copy(ewm.at[j0 + i], deg_sh.at[colm.at[j0 + i]], sd,
                             add=True)
        for i in range(FB):
            pltpu.make_async_copy(ewm.at[j0 + i],
                                  deg_sh.at[colm.at[j0 + i]], sd).wait()
        return carry

    lax.fori_loop(0, NCHUNK // FB, group, 0)
    for j in range(NCHUNK - NCHUNK % FB, NCHUNK):
        pltpu.async_copy(ewm.at[j], deg_sh.at[colm.at[j]], sd, add=True)
    for j in range(NCHUNK - NCHUNK % FB, NCHUNK):
        pltpu.make_async_copy(ewm.at[j], deg_sh.at[colm.at[j]], sd).wait()
    plsc.subcore_barrier()
    pltpu.sync_copy(
        deg_sh.at[pl.ds(rbase, STRIPE)], out_hbm.at[c, pl.ds(rbase, STRIPE)]
    )


def _make_agg(D, pages):
    G = D // 16
    PGMAX = max(p for _, p in pages)

    @functools.partial(
        pl.kernel,
        out_type=jax.ShapeDtypeStruct((NC, NPAD, D), jnp.float32),
        mesh=_mesh,
        scratch_types=[
            pltpu.VMEM((PGMAX, K), jnp.int32),
            pltpu.VMEM((PGMAX, K), jnp.int32),
            pltpu.VMEM((PGMAX, K), jnp.float32),
            pltpu.VMEM((K, D), jnp.float32),
            pltpu.VMEM((K, D), jnp.float32),
            pltpu.VMEM((K, D), jnp.float32),
            pltpu.VMEM((K, D), jnp.float32),
            pltpu.VMEM_SHARED((NPAD, D), jnp.float32),
            pltpu.SemaphoreType.DMA,
            pltpu.SemaphoreType.DMA,
            pltpu.SemaphoreType.DMA,
            pltpu.SemaphoreType.DMA,
            pltpu.SemaphoreType.DMA,
            pltpu.SemaphoreType.DMA,
        ],
        compiler_params=pltpu.CompilerParams(use_tc_tiling_on_sc=False),
    )
    def _agg(h_hbm, row_hbm, col_hbm, ew_hbm, out_hbm, rowm, colm, ewm,
             gb0, gb1, sb0, sb1, acc_sh, sg0, sg1, sh0, sh1, ss0, ss1):
        c = lax.axis_index("c")
        s = lax.axis_index("s")
        wid = s * NC + c
        rbase = s * STRIPE
        cbase = wid * NCHUNK
        pltpu.sync_copy(
            h_hbm.at[pl.ds(rbase, STRIPE)], acc_sh.at[pl.ds(rbase, STRIPE)]
        )
        plsc.subcore_barrier()

        H = K // 2

        def g_start(j, buf, sem, sem2):
            pltpu.async_copy(h_hbm.at[rowm.at[j, pl.ds(0, H)]],
                             buf.at[pl.ds(0, H)], sem)
            pltpu.async_copy(h_hbm.at[rowm.at[j, pl.ds(H, H)]],
                             buf.at[pl.ds(H, H)], sem2)

        def g_wait(j, buf, sem, sem2):
            pltpu.make_async_copy(h_hbm.at[rowm.at[j, pl.ds(0, H)]],
                                  buf.at[pl.ds(0, H)], sem).wait()
            pltpu.make_async_copy(h_hbm.at[rowm.at[j, pl.ds(H, H)]],
                                  buf.at[pl.ds(H, H)], sem2).wait()

        def s_start(j, buf, sem):
            pltpu.async_copy(buf, acc_sh.at[colm.at[j]], sem, add=True)

        def s_wait(j, buf, sem):
            pltpu.make_async_copy(buf, acc_sh.at[colm.at[j]], sem).wait()

        def scale(j, src, dst):
            @plsc.parallel_loop(0, K // 16, unroll=K // 16)
            def srow(t):
                ew16 = ewm[j, pl.ds(t * 16, 16)]
                for kk in range(16):
                    w = ew16[kk]
                    k = t * 16 + kk
                    for g in range(G):
                        dst[k, pl.ds(g * 16, 16)] = (
                            src[k, pl.ds(g * 16, 16)] * w
                        )

        GB = (gb0, gb1)
        SB = (sb0, sb1)
        SG = (sg0, sg1)
        SH = (sh0, sh1)
        SS = (ss0, ss1)

        for c0, P in pages:
            pltpu.sync_copy(row_hbm.at[pl.ds(cbase + c0, P)],
                            rowm.at[pl.ds(0, P)])
            pltpu.sync_copy(col_hbm.at[pl.ds(cbase + c0, P)],
                            colm.at[pl.ds(0, P)])
            pltpu.sync_copy(ew_hbm.at[pl.ds(cbase + c0, P)],
                            ewm.at[pl.ds(0, P)])
            g_start(0, GB[0], SG[0], SH[0])
            g_start(1, GB[1], SG[1], SH[1])

            def step(j, p):
                g_wait(j, GB[p], SG[p], SH[p])

                @pl.when(j >= 2)
                def _():
                    s_wait(j - 2, SB[p], SS[p])

                scale(j, GB[p], SB[p])

                @pl.when(j + 2 < P)
                def _():
                    g_start(j + 2, GB[p], SG[p], SH[p])

                s_start(j, SB[p], SS[p])

            def pair(jj, carry):
                step(2 * jj, 0)
                step(2 * jj + 1, 1)
                return carry

            lax.fori_loop(0, P // 2, pair, 0)
            if P % 2:
                step(P - 1, (P - 1) % 2)
            s_wait(P - 2, SB[(P - 2) % 2], SS[(P - 2) % 2])
            s_wait(P - 1, SB[(P - 1) % 2], SS[(P - 1) % 2])

        plsc.subcore_barrier()
        pltpu.sync_copy(
            acc_sh.at[pl.ds(rbase, STRIPE)], out_hbm.at[c, pl.ds(rbase, STRIPE)]
        )

    return _agg


_agg128 = _make_agg(D_HID, pages=[(0, 32), (32, 32), (64, 32), (96, 29)])
_agg48 = _make_agg(D_OUT_PAD, pages=[(0, NCHUNK)])


BLK = 512
GRID = NPAD // BLK


def _lin1_body(x_ref, w_ref, d0_ref, d1_ref, h_ref, dis_ref):
    deg = d0_ref[...] + d1_ref[...] + 1.0
    dis = jnp.where(deg > 0, lax.rsqrt(deg), 0.0)
    h = lax.dot_general(
        x_ref[...], w_ref[...],
        (((1,), (1,)), ((), ())),
        preferred_element_type=jnp.float32,
    )
    h_ref[...] = h * dis
    dis_ref[...] = dis


def _lin1_call(xp, W1, d0, d1):
    return pl.pallas_call(
        _lin1_body,
        grid=(GRID,),
        in_specs=[
            pl.BlockSpec((BLK, D_IN), lambda i: (i, 0)),
            pl.BlockSpec((D_HID, D_IN), lambda i: (0, 0)),
            pl.BlockSpec((BLK, 1), lambda i: (i, 0)),
            pl.BlockSpec((BLK, 1), lambda i: (i, 0)),
        ],
        out_specs=[
            pl.BlockSpec((BLK, D_HID), lambda i: (i, 0)),
            pl.BlockSpec((BLK, 1), lambda i: (i, 0)),
        ],
        out_shape=[
            jax.ShapeDtypeStruct((NPAD, D_HID), jnp.float32),
            jax.ShapeDtypeStruct((NPAD, 1), jnp.float32),
        ],
    )(xp, W1, d0, d1)


def _mid_body(p0_ref, p1_ref, h1_ref, dis_ref, w2_ref, b1_ref, h2_ref):
    dis = dis_ref[...]
    z = (p0_ref[...] + p1_ref[...] - h1_ref[...]) * dis + b1_ref[...]
    z = jnp.maximum(z, 0.0)
    h2 = lax.dot_general(
        z, w2_ref[...],
        (((1,), (1,)), ((), ())),
        preferred_element_type=jnp.float32,
    )
    h2_ref[...] = h2 * dis


def _mid_call(p0, p1, h1, dis, W2p, b1r):
    return pl.pallas_call(
        _mid_body,
        grid=(GRID,),
        in_specs=[
            pl.BlockSpec((BLK, D_HID), lambda i: (i, 0)),
            pl.BlockSpec((BLK, D_HID), lambda i: (i, 0)),
            pl.BlockSpec((BLK, D_HID), lambda i: (i, 0)),
            pl.BlockSpec((BLK, 1), lambda i: (i, 0)),
            pl.BlockSpec((D_OUT_PAD, D_HID), lambda i: (0, 0)),
            pl.BlockSpec((1, D_HID), lambda i: (0, 0)),
        ],
        out_specs=pl.BlockSpec((BLK, D_OUT_PAD), lambda i: (i, 0)),
        out_shape=jax.ShapeDtypeStruct((NPAD, D_OUT_PAD), jnp.float32),
    )(p0, p1, h1, dis, W2p, b1r)


def _out_body(p0_ref, p1_ref, h2_ref, dis_ref, b2_ref, o_ref):
    o_ref[...] = (
        (p0_ref[...] + p1_ref[...] - h2_ref[...]) * dis_ref[...] + b2_ref[...]
    )


def _out_call(p0, p1, h2, dis, b2r):
    return pl.pallas_call(
        _out_body,
        grid=(GRID,),
        in_specs=[
            pl.BlockSpec((BLK, D_OUT_PAD), lambda i: (i, 0)),
            pl.BlockSpec((BLK, D_OUT_PAD), lambda i: (i, 0)),
            pl.BlockSpec((BLK, D_OUT_PAD), lambda i: (i, 0)),
            pl.BlockSpec((BLK, 1), lambda i: (i, 0)),
            pl.BlockSpec((1, D_OUT_PAD), lambda i: (0, 0)),
        ],
        out_specs=pl.BlockSpec((BLK, D_OUT_PAD), lambda i: (i, 0)),
        out_shape=jax.ShapeDtypeStruct((NPAD, D_OUT_PAD), jnp.float32),
    )(p0, p1, h2, dis, b2r)


def kernel(x, edge_index, edge_weight, W1, b1, W2, b2):
    xp = jnp.pad(x, ((0, NPAD - N), (0, 0)))
    row2 = edge_index[0].reshape(E // K, K)
    col2 = edge_index[1].reshape(E // K, K)
    ew2 = edge_weight.reshape(E // K, K)

    degp = _deg_kernel(col2, ew2)
    d0 = degp[0].reshape(NPAD, 1)
    d1 = degp[1].reshape(NPAD, 1)

    h1, dis = _lin1_call(xp, W1, d0, d1)
    p1 = _agg128(h1, row2, col2, ew2)

    W2p = jnp.pad(W2, ((0, D_OUT_PAD - D_OUT), (0, 0)))
    h2 = _mid_call(p1[0], p1[1], h1, dis, W2p, b1.reshape(1, -1))

    p2 = _agg48(h2, row2, col2, ew2)
    b2p = jnp.pad(b2, (0, D_OUT_PAD - D_OUT)).reshape(1, -1)
    out = _out_call(p2[0], p2[1], h2, dis, b2p)
    return out[:N, :D_OUT]
